# baseline (speedup 1.0000x reference)
import functools

import jax
import jax.numpy as jnp
from jax import lax
from jax.experimental import pallas as pl
from jax.experimental.pallas import tpu as pltpu
from jax.experimental.pallas import tpu_sc as plsc

_DIMS = 1000
_DPAD = 1024
_NWORDS = _DPAD // 32
_LEVELS = 100
_SIZE = 128
_BATCH = 1024
_NTILES = 32
_BPT = _BATCH // _NTILES
_LANES = 16


def _csa(a, b, c):
    t = a ^ b
    return t ^ c, (a & b) | (t & c)


def _accumulate16(carry, planes):
    ones, twos, fours, eights, s16, s32, s64, s128 = carry
    c2 = []
    for p in range(8):
        ones, c = _csa(planes[2 * p], planes[2 * p + 1], ones)
        c2.append(c)
    c4 = []
    for p in range(4):
        twos, c = _csa(c2[2 * p], c2[2 * p + 1], twos)
        c4.append(c)
    c8 = []
    for p in range(2):
        fours, c = _csa(c4[2 * p], c4[2 * p + 1], fours)
        c8.append(c)
    eights, c16 = _csa(c8[0], c8[1], eights)
    s16, c32 = s16 ^ c16, s16 & c16
    s32, c64 = s32 ^ c32, s32 & c32
    s64, c128 = s64 ^ c64, s64 & c64
    s128 = s128 ^ c128
    return ones, twos, fours, eights, s16, s32, s64, s128


def _sc_encode_body(x_hbm, kinv_hbm, vbit_hbm, out_hbm,
                    x_v, idx_v, kinv_v, vbit_v, outstage_v, dma_sem, stage_sem, table_sem):
    wid = lax.axis_index("s") * 2 + lax.axis_index("c")
    _tile_work(wid, x_hbm, kinv_hbm, vbit_hbm, out_hbm,
               x_v, idx_v, kinv_v, vbit_v, outstage_v, dma_sem, stage_sem, table_sem)


def _tile_work(wid, x_hbm, kinv_hbm, vbit_hbm, out_hbm,
               x_v, idx_v, kinv_v, vbit_v, outstage_v, dma_sem, stage_sem, table_sem):
    base = wid * _BPT
    lanes = lax.iota(jnp.int32, _LANES)

    x_copy = pltpu.async_copy(x_hbm.at[pl.ds(base, _BPT)], x_v, stage_sem)
    kinv_copy = pltpu.async_copy(kinv_hbm, kinv_v, table_sem)
    vbit_copy = pltpu.async_copy(vbit_hbm, vbit_v, table_sem)

    x_copy.wait()
    kinv_copy.wait()
    vbit_copy.wait()

    zero = jnp.zeros((_LANES,), jnp.int32)

    @plsc.parallel_loop(0, _BPT, 2)
    def _rowpair(b0):
        for row in range(2):
            for c in range(_SIZE // _LANES):
                xv = x_v[b0 + row, pl.ds(c * _LANES, _LANES)]
                y = jnp.clip(xv, 0.0, 1.0) * jnp.float32(_LEVELS - 1)
                t0 = y.astype(jnp.int32)
                r = y - t0.astype(jnp.float32)
                up = (r > 0.5) | ((r == 0.5) & ((t0 & 1) == 1))
                q = t0 + jnp.where(up, jnp.int32(1), jnp.int32(0))
                idx_v[b0 + row, pl.ds(c * _LANES, _LANES)] = q * jnp.int32(_NWORDS)
        for col in range(2):
            co = col * _LANES
            init = (zero,) * 16

            @pl.loop(0, _SIZE // _LANES, init_carry=init)
            def _groups(gg, carry):
                kws = [kinv_v[pl.ds((gg * _LANES + u) * _NWORDS + co, _LANES)]
                       for u in range(_LANES)]
                new = []
                for row in range(2):
                    iv = idx_v[b0 + row, pl.ds(gg * _LANES, _LANES)]
                    planes = [vbit_v[pl.ds(iv[u] + co, _LANES)] ^ kws[u]
                              for u in range(_LANES)]
                    new.extend(_accumulate16(carry[row * 8:row * 8 + 8],
                                             tuple(planes)))
                return tuple(new)

            acc = _groups
            for row in range(2):
                ones, twos, fours, eights, s16, s32, s64, s128 = \
                    acc[row * 8:row * 8 + 8]
                low_any = ones | twos | fours | eights | s16 | s32
                ge = s128 | (s64 & low_any)
                rb = (b0 + row) * _DPAD
                for k in range(_LANES):
                    wv = jnp.full((_LANES,), ge[k], jnp.int32)
                    lo = (wv >> lanes) & 1
                    hi = (wv >> (lanes + 16)) & 1
                    ww = col * _LANES + k
                    outstage_v[pl.ds(rb + ww * 32, _LANES)] = jnp.where(
                        lo == 1, jnp.float32(1.0), jnp.float32(-1.0))
                    outstage_v[pl.ds(rb + ww * 32 + 16, _LANES)] = jnp.where(
                        hi == 1, jnp.float32(1.0), jnp.float32(-1.0))
        for row in range(2):
            pltpu.async_copy(
                outstage_v.at[pl.ds((b0 + row) * _DPAD, _DIMS)],
                out_hbm.at[pl.ds((base + b0 + row) * _DIMS, _DIMS)],
                dma_sem)

    for b in range(_BPT):
        pltpu.make_async_copy(
            outstage_v.at[pl.ds(b * _DPAD, _DIMS)],
            out_hbm.at[pl.ds((base + b) * _DIMS, _DIMS)],
            dma_sem).wait()


@functools.lru_cache(maxsize=None)
def _sc_encode():
    return functools.partial(
        pl.kernel,
        out_type=jax.ShapeDtypeStruct((_BATCH * _DIMS,), jnp.float32),
        mesh=plsc.VectorSubcoreMesh(core_axis_name="c", subcore_axis_name="s",
                                    num_cores=2, num_subcores=16),
        scratch_types=[
            pltpu.VMEM((_BPT, _SIZE), jnp.float32),
            pltpu.VMEM((_BPT, _SIZE), jnp.int32),
            pltpu.VMEM((_SIZE * _NWORDS,), jnp.int32),
            pltpu.VMEM((_LEVELS * _NWORDS,), jnp.int32),
            pltpu.VMEM((_BPT * _DPAD,), jnp.float32),
            pltpu.SemaphoreType.DMA,
            pltpu.SemaphoreType.DMA,
            pltpu.SemaphoreType.DMA,
        ],
    )(_sc_encode_body)


def _pack_bits(bits):
    r = bits.shape[0]
    padded = jnp.pad(bits, ((0, 0), (0, _DPAD - _DIMS)))
    shifted = padded.reshape(r, _NWORDS, 32).astype(jnp.uint32) << jnp.arange(
        32, dtype=jnp.uint32)
    words = jnp.sum(shifted, axis=-1, dtype=jnp.uint32)
    return lax.bitcast_convert_type(words, jnp.int32)


@jax.jit
def kernel(x, keys_weight, values_weight):
    kinv = _pack_bits(keys_weight <= 0).reshape(-1)
    vbit = _pack_bits(values_weight > 0).reshape(-1)
    out = _sc_encode()(x, kinv, vbit)
    return out.reshape(_BATCH, _DIMS)


# speedup vs baseline: 1.0371x; 1.0371x over previous
import functools

import jax
import jax.numpy as jnp
from jax import lax
from jax.experimental import pallas as pl
from jax.experimental.pallas import tpu as pltpu
from jax.experimental.pallas import tpu_sc as plsc

_DIMS = 1000
_DPAD = 1024
_NWORDS = _DPAD // 32
_LEVELS = 100
_SIZE = 128
_BATCH = 1024
_NTILES = 32
_BPT = _BATCH // _NTILES
_LANES = 16


def _csa(a, b, c):
    t = a ^ b
    return t ^ c, (a & b) | (t & c)


def _accumulate16(carry, planes):
    ones, twos, fours, eights, s16, s32, s64, s128 = carry
    c2 = []
    for p in range(8):
        ones, c = _csa(planes[2 * p], planes[2 * p + 1], ones)
        c2.append(c)
    c4 = []
    for p in range(4):
        twos, c = _csa(c2[2 * p], c2[2 * p + 1], twos)
        c4.append(c)
    c8 = []
    for p in range(2):
        fours, c = _csa(c4[2 * p], c4[2 * p + 1], fours)
        c8.append(c)
    eights, c16 = _csa(c8[0], c8[1], eights)
    s16, c32 = s16 ^ c16, s16 & c16
    s32, c64 = s32 ^ c32, s32 & c32
    s64, c128 = s64 ^ c64, s64 & c64
    s128 = s128 ^ c128
    return ones, twos, fours, eights, s16, s32, s64, s128


def _sc_encode_body(x_hbm, kinv_hbm, vbit_hbm, out_hbm,
                    x_v, idx_v, kinv_v, vbit_v, outstage_v, dma_sem, stage_sem, table_sem):
    wid = lax.axis_index("s") * 2 + lax.axis_index("c")
    _tile_work(wid, x_hbm, kinv_hbm, vbit_hbm, out_hbm,
               x_v, idx_v, kinv_v, vbit_v, outstage_v, dma_sem, stage_sem, table_sem)


def _tile_work(wid, x_hbm, kinv_hbm, vbit_hbm, out_hbm,
               x_v, idx_v, kinv_v, vbit_v, outstage_v, dma_sem, stage_sem, table_sem):
    base = wid * _BPT
    lanes = lax.iota(jnp.int32, _LANES)

    x_copy = pltpu.async_copy(x_hbm.at[pl.ds(base, _BPT)], x_v, stage_sem)
    kinv_copy = pltpu.async_copy(kinv_hbm, kinv_v, table_sem)
    vbit_copy = pltpu.async_copy(vbit_hbm, vbit_v, table_sem)
    x_copy.wait()

    @pl.loop(0, _BPT)
    def _quant(b):
        for c in range(_SIZE // _LANES):
            xv = x_v[b, pl.ds(c * _LANES, _LANES)]
            y = jnp.clip(xv, 0.0, 1.0) * jnp.float32(_LEVELS - 1)
            t0 = y.astype(jnp.int32)
            r = y - t0.astype(jnp.float32)
            up = (r > 0.5) | ((r == 0.5) & ((t0 & 1) == 1))
            q = t0 + jnp.where(up, jnp.int32(1), jnp.int32(0))
            idx_v[b, pl.ds(c * _LANES, _LANES)] = q * jnp.int32(_NWORDS)

    kinv_copy.wait()
    vbit_copy.wait()

    zero = jnp.zeros((_LANES,), jnp.int32)

    @plsc.parallel_loop(0, _BPT, 2)
    def _rowpair(b0):
        for col in range(2):
            co = col * _LANES
            init = (zero,) * 16

            @pl.loop(0, _SIZE // _LANES, init_carry=init, unroll=2)
            def _groups(gg, carry):
                kws = [kinv_v[pl.ds((gg * _LANES + u) * _NWORDS + co, _LANES)]
                       for u in range(_LANES)]
                new = []
                for row in range(2):
                    iv = idx_v[b0 + row, pl.ds(gg * _LANES, _LANES)]
                    planes = [vbit_v[pl.ds(iv[u] + co, _LANES)] ^ kws[u]
                              for u in range(_LANES)]
                    new.extend(_accumulate16(carry[row * 8:row * 8 + 8],
                                             tuple(planes)))
                return tuple(new)

            acc = _groups
            for row in range(2):
                ones, twos, fours, eights, s16, s32, s64, s128 = \
                    acc[row * 8:row * 8 + 8]
                low_any = ones | twos | fours | eights | s16 | s32
                ge = s128 | (s64 & low_any)
                rb = (b0 + row) * _DPAD
                for k in range(_LANES):
                    wv = jnp.full((_LANES,), ge[k], jnp.int32)
                    lo = (wv >> lanes) & 1
                    hi = (wv >> (lanes + 16)) & 1
                    ww = col * _LANES + k
                    outstage_v[pl.ds(rb + ww * 32, _LANES)] = jnp.where(
                        lo == 1, jnp.float32(1.0), jnp.float32(-1.0))
                    outstage_v[pl.ds(rb + ww * 32 + 16, _LANES)] = jnp.where(
                        hi == 1, jnp.float32(1.0), jnp.float32(-1.0))
        for row in range(2):
            pltpu.async_copy(
                outstage_v.at[pl.ds((b0 + row) * _DPAD, _DIMS)],
                out_hbm.at[pl.ds((base + b0 + row) * _DIMS, _DIMS)],
                dma_sem)

    for b in range(_BPT):
        pltpu.make_async_copy(
            outstage_v.at[pl.ds(b * _DPAD, _DIMS)],
            out_hbm.at[pl.ds((base + b) * _DIMS, _DIMS)],
            dma_sem).wait()


@functools.lru_cache(maxsize=None)
def _sc_encode():
    return functools.partial(
        pl.kernel,
        out_type=jax.ShapeDtypeStruct((_BATCH * _DIMS,), jnp.float32),
        mesh=plsc.VectorSubcoreMesh(core_axis_name="c", subcore_axis_name="s",
                                    num_cores=2, num_subcores=16),
        scratch_types=[
            pltpu.VMEM((_BPT, _SIZE), jnp.float32),
            pltpu.VMEM((_BPT, _SIZE), jnp.int32),
            pltpu.VMEM((_SIZE * _NWORDS,), jnp.int32),
            pltpu.VMEM((_LEVELS * _NWORDS,), jnp.int32),
            pltpu.VMEM((_BPT * _DPAD,), jnp.float32),
            pltpu.SemaphoreType.DMA,
            pltpu.SemaphoreType.DMA,
            pltpu.SemaphoreType.DMA,
        ],
    )(_sc_encode_body)


def _pack_bits(bits):
    r = bits.shape[0]
    padded = jnp.pad(bits, ((0, 0), (0, _DPAD - _DIMS)))
    shifted = padded.reshape(r, _NWORDS, 32).astype(jnp.uint32) << jnp.arange(
        32, dtype=jnp.uint32)
    words = jnp.sum(shifted, axis=-1, dtype=jnp.uint32)
    return lax.bitcast_convert_type(words, jnp.int32)


@jax.jit
def kernel(x, keys_weight, values_weight):
    kinv = _pack_bits(keys_weight <= 0).reshape(-1)
    vbit = _pack_bits(values_weight > 0).reshape(-1)
    out = _sc_encode()(x, kinv, vbit)
    return out.reshape(_BATCH, _DIMS)


# speedup vs baseline: 1.0484x; 1.0109x over previous
import functools

import jax
import jax.numpy as jnp
from jax import lax
from jax.experimental import pallas as pl
from jax.experimental.pallas import tpu as pltpu
from jax.experimental.pallas import tpu_sc as plsc

_DIMS = 1000
_DPAD = 1024
_NWORDS = _DPAD // 32
_LEVELS = 100
_SIZE = 128
_BATCH = 1024
_NTILES = 32
_BPT = _BATCH // _NTILES
_LANES = 16


def _csa(a, b, c):
    t = a ^ b
    return t ^ c, (a & b) | (t & c)


def _accumulate16(carry, planes):
    ones, twos, fours, eights, s16, s32, s64, s128 = carry
    c2 = []
    for p in range(8):
        ones, c = _csa(planes[2 * p], planes[2 * p + 1], ones)
        c2.append(c)
    c4 = []
    for p in range(4):
        twos, c = _csa(c2[2 * p], c2[2 * p + 1], twos)
        c4.append(c)
    c8 = []
    for p in range(2):
        fours, c = _csa(c4[2 * p], c4[2 * p + 1], fours)
        c8.append(c)
    eights, c16 = _csa(c8[0], c8[1], eights)
    s16, c32 = s16 ^ c16, s16 & c16
    s32, c64 = s32 ^ c32, s32 & c32
    s64, c128 = s64 ^ c64, s64 & c64
    s128 = s128 ^ c128
    return ones, twos, fours, eights, s16, s32, s64, s128


def _sc_encode_body(x_hbm, kinv_hbm, vbit_hbm, out_hbm,
                    x_v, idx_v, kinv_v, vbit_v, outstage_v, dma_sem, stage_sem, table_sem):
    wid = lax.axis_index("s") * 2 + lax.axis_index("c")
    _tile_work(wid, x_hbm, kinv_hbm, vbit_hbm, out_hbm,
               x_v, idx_v, kinv_v, vbit_v, outstage_v, dma_sem, stage_sem, table_sem)


def _tile_work(wid, x_hbm, kinv_hbm, vbit_hbm, out_hbm,
               x_v, idx_v, kinv_v, vbit_v, outstage_v, dma_sem, stage_sem, table_sem):
    base = wid * _BPT
    lanes = lax.iota(jnp.int32, _LANES)

    x_copy = pltpu.async_copy(x_hbm.at[pl.ds(base, _BPT)], x_v, stage_sem)
    kinv_copy = pltpu.async_copy(kinv_hbm, kinv_v, table_sem)
    vbit_copy = pltpu.async_copy(vbit_hbm, vbit_v, table_sem)
    x_copy.wait()

    @pl.loop(0, _BPT)
    def _quant(b):
        for c in range(_SIZE // _LANES):
            xv = x_v[b, pl.ds(c * _LANES, _LANES)]
            y = jnp.clip(xv, 0.0, 1.0) * jnp.float32(_LEVELS - 1)
            t0 = y.astype(jnp.int32)
            r = y - t0.astype(jnp.float32)
            up = (r > 0.5) | ((r == 0.5) & ((t0 & 1) == 1))
            q = t0 + jnp.where(up, jnp.int32(1), jnp.int32(0))
            idx_v[b, pl.ds(c * _LANES, _LANES)] = q * jnp.int32(_NWORDS)

    kinv_copy.wait()
    vbit_copy.wait()

    zero = jnp.zeros((_LANES,), jnp.int32)

    @plsc.parallel_loop(0, _BPT, 2)
    def _rowpair(b0):
        for col in range(2):
            co = col * _LANES
            init = (zero,) * 16

            @pl.loop(0, _SIZE // _LANES, init_carry=init, unroll=4)
            def _groups(gg, carry):
                kws = [kinv_v[pl.ds((gg * _LANES + u) * _NWORDS + co, _LANES)]
                       for u in range(_LANES)]
                new = []
                for row in range(2):
                    iv = idx_v[b0 + row, pl.ds(gg * _LANES, _LANES)]
                    planes = [vbit_v[pl.ds(iv[u] + co, _LANES)] ^ kws[u]
                              for u in range(_LANES)]
                    new.extend(_accumulate16(carry[row * 8:row * 8 + 8],
                                             tuple(planes)))
                return tuple(new)

            acc = _groups
            for row in range(2):
                ones, twos, fours, eights, s16, s32, s64, s128 = \
                    acc[row * 8:row * 8 + 8]
                low_any = ones | twos | fours | eights | s16 | s32
                ge = s128 | (s64 & low_any)
                rb = (b0 + row) * _DPAD
                for k in range(_LANES):
                    wv = jnp.full((_LANES,), ge[k], jnp.int32)
                    lo = (wv >> lanes) & 1
                    hi = (wv >> (lanes + 16)) & 1
                    ww = col * _LANES + k
                    outstage_v[pl.ds(rb + ww * 32, _LANES)] = jnp.where(
                        lo == 1, jnp.float32(1.0), jnp.float32(-1.0))
                    outstage_v[pl.ds(rb + ww * 32 + 16, _LANES)] = jnp.where(
                        hi == 1, jnp.float32(1.0), jnp.float32(-1.0))
        for row in range(2):
            pltpu.async_copy(
                outstage_v.at[pl.ds((b0 + row) * _DPAD, _DIMS)],
                out_hbm.at[pl.ds((base + b0 + row) * _DIMS, _DIMS)],
                dma_sem)

    for b in range(_BPT):
        pltpu.make_async_copy(
            outstage_v.at[pl.ds(b * _DPAD, _DIMS)],
            out_hbm.at[pl.ds((base + b) * _DIMS, _DIMS)],
            dma_sem).wait()


@functools.lru_cache(maxsize=None)
def _sc_encode():
    return functools.partial(
        pl.kernel,
        out_type=jax.ShapeDtypeStruct((_BATCH * _DIMS,), jnp.float32),
        mesh=plsc.VectorSubcoreMesh(core_axis_name="c", subcore_axis_name="s",
                                    num_cores=2, num_subcores=16),
        scratch_types=[
            pltpu.VMEM((_BPT, _SIZE), jnp.float32),
            pltpu.VMEM((_BPT, _SIZE), jnp.int32),
            pltpu.VMEM((_SIZE * _NWORDS,), jnp.int32),
            pltpu.VMEM((_LEVELS * _NWORDS,), jnp.int32),
            pltpu.VMEM((_BPT * _DPAD,), jnp.float32),
            pltpu.SemaphoreType.DMA,
            pltpu.SemaphoreType.DMA,
            pltpu.SemaphoreType.DMA,
        ],
    )(_sc_encode_body)


def _pack_bits(bits):
    r = bits.shape[0]
    padded = jnp.pad(bits, ((0, 0), (0, _DPAD - _DIMS)))
    shifted = padded.reshape(r, _NWORDS, 32).astype(jnp.uint32) << jnp.arange(
        32, dtype=jnp.uint32)
    words = jnp.sum(shifted, axis=-1, dtype=jnp.uint32)
    return lax.bitcast_convert_type(words, jnp.int32)


@jax.jit
def kernel(x, keys_weight, values_weight):
    kinv = _pack_bits(keys_weight <= 0).reshape(-1)
    vbit = _pack_bits(values_weight > 0).reshape(-1)
    out = _sc_encode()(x, kinv, vbit)
    return out.reshape(_BATCH, _DIMS)
